# Initial kernel scaffold; baseline (speedup 1.0000x reference)
import jax
import jax.numpy as jnp
from jax.experimental import pallas as pl


def kernel(feat, edge_index, W, b):
    raise NotImplementedError("write your pallas kernel here")


# speedup vs baseline: 5.4235x; 5.4235x over previous
import functools

import jax
import jax.numpy as jnp
from jax import lax
from jax.experimental import pallas as pl
from jax.experimental.pallas import tpu as pltpu
from jax.experimental.pallas import tpu_sc as plsc

N = 10000
E = 320000
D = 128
HD = D // 2

NC = 2
NS = 16
NW = NC * NS
C = 80
DCH = E // NW // C
HCH = E // NS // C
NPAD = 10240
RPS = NPAD // NS
DC = 128
NDC = RPS // DC

_MESH = dict(core_axis_name="c", subcore_axis_name="s", num_cores=NC,
             num_subcores=NS)


@functools.partial(
    pl.kernel,
    out_type=jax.ShapeDtypeStruct((NC * NPAD,), jnp.float32),
    mesh=plsc.VectorSubcoreMesh(**_MESH),
    scratch_types=[
        pltpu.VMEM((DCH, C), jnp.int32),
        pltpu.VMEM((C,), jnp.float32),
        pltpu.VMEM((RPS,), jnp.float32),
        pltpu.VMEM_SHARED((NPAD,), jnp.float32),
    ],
)
def _deg_kernel(dst_hbm, out_hbm, dst_v, ones_v, stage_v, acc):
    c = lax.axis_index("c")
    s = lax.axis_index("s")
    wid = s * NC + c

    one16 = jnp.full((16,), 1.0, jnp.float32)
    zero16 = jnp.zeros((16,), jnp.float32)
    for t in range(C // 16):
        ones_v[pl.ds(t * 16, 16)] = one16

    def zfill(i, carry):
        stage_v[pl.ds(i * 16, 16)] = zero16
        return carry

    lax.fori_loop(0, RPS // 16, zfill, 0)
    pltpu.sync_copy(stage_v, acc.at[pl.ds(s * RPS, RPS)])
    pltpu.sync_copy(dst_hbm.at[wid], dst_v)
    plsc.subcore_barrier()

    def body(j, carry):
        pltpu.sync_copy(ones_v, acc.at[dst_v.at[j]], add=True)
        return carry

    lax.fori_loop(0, DCH, body, 0)
    plsc.subcore_barrier()

    pltpu.sync_copy(acc.at[pl.ds(s * RPS, RPS)], stage_v)
    pltpu.sync_copy(stage_v, out_hbm.at[pl.ds(c * NPAD + s * RPS, RPS)])


@functools.partial(
    pl.kernel,
    out_type=jax.ShapeDtypeStruct((NC * NPAD, HD), jnp.float32),
    mesh=plsc.VectorSubcoreMesh(**_MESH),
    scratch_types=[
        pltpu.VMEM((HCH, C), jnp.int32),
        pltpu.VMEM((HCH, C), jnp.int32),
        pltpu.VMEM((C, HD), jnp.float32),
        pltpu.VMEM((DC, HD), jnp.float32),
        pltpu.VMEM_SHARED((NPAD, HD), jnp.float32),
        pltpu.SemaphoreType.DMA,
    ],
    compiler_params=pltpu.CompilerParams(use_tc_tiling_on_sc=False),
)
def _hop_kernel(g_hbm, src_hbm, dst_hbm, out_hbm, src_v, dst_v, rows_v,
                stage_v, acc, sem):
    c = lax.axis_index("c")
    s = lax.axis_index("s")

    zero16 = jnp.zeros((16,), jnp.float32)

    def zrow(r, carry):
        for t in range(HD // 16):
            stage_v[r, pl.ds(t * 16, 16)] = zero16
        return carry

    lax.fori_loop(0, DC, zrow, 0)
    for k in range(NDC):
        pltpu.sync_copy(stage_v, acc.at[pl.ds(s * RPS + k * DC, DC)])
    pltpu.sync_copy(src_hbm.at[s], src_v)
    pltpu.sync_copy(dst_hbm.at[s], dst_v)

    off16 = jnp.zeros((16,), jnp.int32) + c * N

    def shift(r, carry):
        for t in range(C // 16):
            src_v[r, pl.ds(t * 16, 16)] = src_v[r, pl.ds(t * 16, 16)] + off16
        return carry

    lax.fori_loop(0, HCH, shift, 0)
    plsc.subcore_barrier()

    def body(j, carry):
        pltpu.async_copy(g_hbm.at[src_v.at[j]], rows_v, sem).wait()
        pltpu.sync_copy(rows_v, acc.at[dst_v.at[j]], add=True)
        return carry

    lax.fori_loop(0, HCH, body, 0)
    plsc.subcore_barrier()

    for k in range(NDC):
        pltpu.sync_copy(acc.at[pl.ds(s * RPS + k * DC, DC)], stage_v)
        pltpu.sync_copy(
            stage_v, out_hbm.at[pl.ds(c * NPAD + s * RPS + k * DC, DC)])


_R = 1000


def _mm_body(x_ref, w_ref, d_ref, o_ref):
    deg = jnp.maximum(d_ref[0] + d_ref[1], 1.0)
    res = jnp.dot(x_ref[...], w_ref[...],
                  preferred_element_type=jnp.float32) * lax.rsqrt(deg)
    o_ref[0] = res[:, :HD]
    o_ref[1] = res[:, HD:]


def _comb_body(p_ref, d_ref, o_ref):
    deg = jnp.maximum(d_ref[0] + d_ref[1], 1.0)
    o_ref[...] = p_ref[...] / deg


def _fin_body(q_ref, d_ref, b_ref, o_ref):
    deg = jnp.maximum(d_ref[0] + d_ref[1], 1.0)
    norm = lax.rsqrt(deg)
    o_ref[...] = (jnp.concatenate([q_ref[0], q_ref[1]], axis=1) * norm
                  + b_ref[...])


def _scaled_mm(feat, W, d3):
    return pl.pallas_call(
        _mm_body,
        grid=(N // _R,),
        in_specs=[
            pl.BlockSpec((_R, D), lambda i: (i, 0)),
            pl.BlockSpec((D, D), lambda i: (0, 0)),
            pl.BlockSpec((NC, _R, 1), lambda i: (0, i, 0)),
        ],
        out_specs=pl.BlockSpec((NC, _R, HD), lambda i: (0, i, 0)),
        out_shape=jax.ShapeDtypeStruct((NC, N, HD), jnp.float32),
    )(feat, W, d3)


def _combine(p, d3):
    return pl.pallas_call(
        _comb_body,
        grid=(N // _R,),
        in_specs=[
            pl.BlockSpec((NC, _R, HD), lambda i: (0, i, 0)),
            pl.BlockSpec((NC, _R, 1), lambda i: (0, i, 0)),
        ],
        out_specs=pl.BlockSpec((NC, _R, HD), lambda i: (0, i, 0)),
        out_shape=jax.ShapeDtypeStruct((NC, N, HD), jnp.float32),
    )(p, d3)


def _finalize(q, d3, b2):
    return pl.pallas_call(
        _fin_body,
        grid=(N // _R,),
        in_specs=[
            pl.BlockSpec((NC, _R, HD), lambda i: (0, i, 0)),
            pl.BlockSpec((NC, _R, 1), lambda i: (0, i, 0)),
            pl.BlockSpec((1, D), lambda i: (0, 0)),
        ],
        out_specs=pl.BlockSpec((_R, D), lambda i: (i, 0)),
        out_shape=jax.ShapeDtypeStruct((N, D), jnp.float32),
    )(q, d3, b2)


def kernel(feat, edge_index, W, b):
    src = edge_index[0].reshape(NS, HCH, C)
    dst_h = edge_index[1].reshape(NS, HCH, C)
    dst_d = edge_index[1].reshape(NW, DCH, C)
    dpart = _deg_kernel(dst_d)
    d3 = dpart.reshape(NC, NPAD, 1)
    g0 = _scaled_mm(feat, W, d3)
    p = _hop_kernel(g0.reshape(NC * N, HD), src, dst_h)
    g1 = _combine(p.reshape(NC, NPAD, HD), d3)
    q = _hop_kernel(g1.reshape(NC * N, HD), src, dst_h)
    return _finalize(q.reshape(NC, NPAD, HD), d3, b.reshape(1, D))


# speedup vs baseline: 8.6206x; 1.5895x over previous
import functools

import jax
import jax.numpy as jnp
from jax import lax
from jax.experimental import pallas as pl
from jax.experimental.pallas import tpu as pltpu
from jax.experimental.pallas import tpu_sc as plsc

N = 10000
E = 320000
D = 128
HD = D // 2

NC = 2
NS = 16
NW = NC * NS
C = 80
DCH = E // NW // C
HCH = E // NS // C
NPAD = 10240
RPS = NPAD // NS
DC = 128
NDC = RPS // DC

_MESH = dict(core_axis_name="c", subcore_axis_name="s", num_cores=NC,
             num_subcores=NS)


@functools.partial(
    pl.kernel,
    out_type=jax.ShapeDtypeStruct((NC * NPAD,), jnp.float32),
    mesh=plsc.VectorSubcoreMesh(**_MESH),
    scratch_types=[
        pltpu.VMEM((DCH, C), jnp.int32),
        pltpu.VMEM((C,), jnp.float32),
        pltpu.VMEM((RPS,), jnp.float32),
        pltpu.VMEM_SHARED((NPAD,), jnp.float32),
    ],
)
def _deg_kernel(dst_hbm, out_hbm, dst_v, ones_v, stage_v, acc):
    c = lax.axis_index("c")
    s = lax.axis_index("s")
    wid = s * NC + c

    one16 = jnp.full((16,), 1.0, jnp.float32)
    zero16 = jnp.zeros((16,), jnp.float32)
    for t in range(C // 16):
        ones_v[pl.ds(t * 16, 16)] = one16

    def zfill(i, carry):
        stage_v[pl.ds(i * 16, 16)] = zero16
        return carry

    lax.fori_loop(0, RPS // 16, zfill, 0)
    pltpu.sync_copy(stage_v, acc.at[pl.ds(s * RPS, RPS)])
    pltpu.sync_copy(dst_hbm.at[wid], dst_v)
    plsc.subcore_barrier()

    def body(j, carry):
        pltpu.sync_copy(ones_v, acc.at[dst_v.at[j]], add=True)
        return carry

    lax.fori_loop(0, DCH, body, 0)
    plsc.subcore_barrier()

    pltpu.sync_copy(acc.at[pl.ds(s * RPS, RPS)], stage_v)
    pltpu.sync_copy(stage_v, out_hbm.at[pl.ds(c * NPAD + s * RPS, RPS)])


@functools.partial(
    pl.kernel,
    out_type=jax.ShapeDtypeStruct((NC * NPAD, HD), jnp.float32),
    mesh=plsc.VectorSubcoreMesh(**_MESH),
    scratch_types=[
        pltpu.VMEM((HCH, C), jnp.int32),
        pltpu.VMEM((HCH, C), jnp.int32),
        pltpu.VMEM((C, HD), jnp.float32),
        pltpu.VMEM((C, HD), jnp.float32),
        pltpu.VMEM((DC, HD), jnp.float32),
        pltpu.VMEM_SHARED((NPAD, HD), jnp.float32),
        pltpu.SemaphoreType.DMA,
        pltpu.SemaphoreType.DMA,
    ],
    compiler_params=pltpu.CompilerParams(use_tc_tiling_on_sc=False),
)
def _hop_kernel(g_hbm, src_hbm, dst_hbm, out_hbm, src_v, dst_v, rows0,
                rows1, stage_v, acc, sem0, sem1):
    c = lax.axis_index("c")
    s = lax.axis_index("s")

    zero16 = jnp.zeros((16,), jnp.float32)

    def zrow(r, carry):
        for t in range(HD // 16):
            stage_v[r, pl.ds(t * 16, 16)] = zero16
        return carry

    lax.fori_loop(0, DC, zrow, 0)
    for k in range(NDC):
        pltpu.sync_copy(stage_v, acc.at[pl.ds(s * RPS + k * DC, DC)])
    pltpu.sync_copy(src_hbm.at[s], src_v)
    pltpu.sync_copy(dst_hbm.at[s], dst_v)

    off16 = jnp.zeros((16,), jnp.int32) + c * N

    def shift(r, carry):
        for t in range(C // 16):
            src_v[r, pl.ds(t * 16, 16)] = src_v[r, pl.ds(t * 16, 16)] + off16
        return carry

    lax.fori_loop(0, HCH, shift, 0)
    plsc.subcore_barrier()

    pltpu.async_copy(g_hbm.at[src_v.at[0]], rows0, sem0)

    def body(i, carry):
        a = 2 * i
        pltpu.async_copy(g_hbm.at[src_v.at[a + 1]], rows1, sem1)
        pltpu.make_async_copy(g_hbm.at[src_v.at[a]], rows0, sem0).wait()
        pltpu.sync_copy(rows0, acc.at[dst_v.at[a]], add=True)
        nxt = jnp.minimum(a + 2, HCH - 1)
        pltpu.async_copy(g_hbm.at[src_v.at[nxt]], rows0, sem0)
        pltpu.make_async_copy(g_hbm.at[src_v.at[a + 1]], rows1, sem1).wait()
        pltpu.sync_copy(rows1, acc.at[dst_v.at[a + 1]], add=True)
        return carry

    lax.fori_loop(0, HCH // 2, body, 0)
    pltpu.make_async_copy(g_hbm.at[src_v.at[HCH - 1]], rows0, sem0).wait()
    plsc.subcore_barrier()

    for k in range(NDC):
        pltpu.sync_copy(acc.at[pl.ds(s * RPS + k * DC, DC)], stage_v)
        pltpu.sync_copy(
            stage_v, out_hbm.at[pl.ds(c * NPAD + s * RPS + k * DC, DC)])


_R = 1000


def _mm_body(x_ref, w_ref, d_ref, o_ref):
    deg = jnp.maximum(d_ref[0] + d_ref[1], 1.0)
    res = jnp.dot(x_ref[...], w_ref[...],
                  preferred_element_type=jnp.float32) * lax.rsqrt(deg)
    o_ref[0] = res[:, :HD]
    o_ref[1] = res[:, HD:]


def _comb_body(p_ref, d_ref, o_ref):
    deg = jnp.maximum(d_ref[0] + d_ref[1], 1.0)
    o_ref[...] = p_ref[...] / deg


def _fin_body(q_ref, d_ref, b_ref, o_ref):
    deg = jnp.maximum(d_ref[0] + d_ref[1], 1.0)
    norm = lax.rsqrt(deg)
    o_ref[...] = (jnp.concatenate([q_ref[0], q_ref[1]], axis=1) * norm
                  + b_ref[...])


def _scaled_mm(feat, W, d3):
    return pl.pallas_call(
        _mm_body,
        grid=(N // _R,),
        in_specs=[
            pl.BlockSpec((_R, D), lambda i: (i, 0)),
            pl.BlockSpec((D, D), lambda i: (0, 0)),
            pl.BlockSpec((NC, _R, 1), lambda i: (0, i, 0)),
        ],
        out_specs=pl.BlockSpec((NC, _R, HD), lambda i: (0, i, 0)),
        out_shape=jax.ShapeDtypeStruct((NC, N, HD), jnp.float32),
    )(feat, W, d3)


def _combine(p, d3):
    return pl.pallas_call(
        _comb_body,
        grid=(N // _R,),
        in_specs=[
            pl.BlockSpec((NC, _R, HD), lambda i: (0, i, 0)),
            pl.BlockSpec((NC, _R, 1), lambda i: (0, i, 0)),
        ],
        out_specs=pl.BlockSpec((NC, _R, HD), lambda i: (0, i, 0)),
        out_shape=jax.ShapeDtypeStruct((NC, N, HD), jnp.float32),
    )(p, d3)


def _finalize(q, d3, b2):
    return pl.pallas_call(
        _fin_body,
        grid=(N // _R,),
        in_specs=[
            pl.BlockSpec((NC, _R, HD), lambda i: (0, i, 0)),
            pl.BlockSpec((NC, _R, 1), lambda i: (0, i, 0)),
            pl.BlockSpec((1, D), lambda i: (0, 0)),
        ],
        out_specs=pl.BlockSpec((_R, D), lambda i: (i, 0)),
        out_shape=jax.ShapeDtypeStruct((N, D), jnp.float32),
    )(q, d3, b2)


def kernel(feat, edge_index, W, b):
    src = edge_index[0].reshape(NS, HCH, C)
    dst_h = edge_index[1].reshape(NS, HCH, C)
    dst_d = edge_index[1].reshape(NW, DCH, C)
    dpart = _deg_kernel(dst_d)
    d3 = dpart.reshape(NC, NPAD, 1)
    g0 = _scaled_mm(feat, W, d3)
    p = _hop_kernel(g0.reshape(NC * N, HD), src, dst_h)
    g1 = _combine(p.reshape(NC, NPAD, HD), d3)
    q = _hop_kernel(g1.reshape(NC * N, HD), src, dst_h)
    return _finalize(q.reshape(NC, NPAD, HD), d3, b.reshape(1, D))
